# baseline (speedup 1.0000x reference)
import functools

import jax
import jax.numpy as jnp
from jax import lax
from jax.experimental import pallas as pl
from jax.experimental.pallas import tpu as pltpu, tpu_sc as plsc

_SC_PARAMS = pltpu.CompilerParams(use_tc_tiling_on_sc=False,
                                  needs_layout_passes=False)
_NW = 32
_LANES = 16


def _wid():
    return lax.axis_index("s") * 2 + lax.axis_index("c")


def _mesh():
    return plsc.VectorSubcoreMesh(core_axis_name="c", subcore_axis_name="s")


def _node_body(x_ref, whu_ref, whw_ref, wih_ref, bih_ref,
               xu_ref, xw_ref, gx_ref):
    x = x_ref[...]
    cdims = (((1,), (1,)), ((), ()))
    xu_ref[...] = lax.dot_general(x, whu_ref[...], cdims,
                                  preferred_element_type=jnp.float32)
    xw_ref[...] = lax.dot_general(x, whw_ref[...], cdims,
                                  preferred_element_type=jnp.float32)
    gx_ref[...] = lax.dot_general(x, wih_ref[...], cdims,
                                  preferred_element_type=jnp.float32) + bih_ref[...]


def _stage_a(x, W_hu, W_hw, W_ih, b_ih):
    n, d = x.shape
    bn = 2000
    grid = (n // bn,)
    return pl.pallas_call(
        _node_body,
        grid=grid,
        in_specs=[
            pl.BlockSpec((bn, d), lambda i: (i, 0)),
            pl.BlockSpec((d, d), lambda i: (0, 0)),
            pl.BlockSpec((d, d), lambda i: (0, 0)),
            pl.BlockSpec((3 * d, d), lambda i: (0, 0)),
            pl.BlockSpec((1, 3 * d), lambda i: (0, 0)),
        ],
        out_specs=[
            pl.BlockSpec((bn, d), lambda i: (i, 0)),
            pl.BlockSpec((bn, d), lambda i: (i, 0)),
            pl.BlockSpec((bn, 3 * d), lambda i: (i, 0)),
        ],
        out_shape=[
            jax.ShapeDtypeStruct((n, d), jnp.float32),
            jax.ShapeDtypeStruct((n, d), jnp.float32),
            jax.ShapeDtypeStruct((n, 3 * d), jnp.float32),
        ],
    )(x, W_hu, W_hw, W_ih, b_ih.reshape(1, 3 * d))


def _epart_body(ea_ref, we_ref, out_ref):
    out_ref[...] = lax.dot_general(ea_ref[...], we_ref[...],
                                   (((1,), (1,)), ((), ())),
                                   preferred_element_type=jnp.float32)


def _stage_a2(edge_attr, W_e):
    e, de = edge_attr.shape
    d = W_e.shape[0]
    be = 10000
    return pl.pallas_call(
        _epart_body,
        grid=(e // be,),
        in_specs=[
            pl.BlockSpec((be, de), lambda i: (i, 0)),
            pl.BlockSpec((d, de), lambda i: (0, 0)),
        ],
        out_specs=pl.BlockSpec((be, d), lambda i: (i, 0)),
        out_shape=jax.ShapeDtypeStruct((e, d), jnp.float32),
    )(edge_attr, W_e)


def _stage_b(e_part, xu, xw, src, tgt):
    E, d = e_part.shape
    epw = E // _NW
    CB = 80
    nch = epw // CB

    @functools.partial(
        pl.kernel,
        out_type=[
            jax.ShapeDtypeStruct((E, d), jnp.float32),
            jax.ShapeDtypeStruct((E, _LANES), jnp.int32),
            jax.ShapeDtypeStruct((E, _LANES), jnp.int32),
        ],
        mesh=_mesh(),
        compiler_params=_SC_PARAMS,
        scratch_types=[
            pltpu.VMEM((CB,), jnp.int32),
            pltpu.VMEM((CB,), jnp.int32),
            pltpu.VMEM((CB, d), jnp.float32),
            pltpu.VMEM((CB, d), jnp.float32),
            pltpu.VMEM((CB, d), jnp.float32),
            pltpu.VMEM((CB, _LANES), jnp.int32),
            pltpu.VMEM((CB, _LANES), jnp.int32),
            pltpu.SemaphoreType.DMA,
            pltpu.SemaphoreType.DMA,
        ],
    )
    def k(ep_hbm, xu_hbm, xw_hbm, src_hbm, tgt_hbm,
          msg_hbm, srep_hbm, trep_hbm,
          sidx, tidx, ebuf, abuf, bbuf, sbro, tbro, sem1, sem2):
        base = _wid() * epw

        def chunk(i, carry):
            e0 = base + i * CB
            pltpu.sync_copy(src_hbm.at[pl.ds(e0, CB)], sidx)
            pltpu.sync_copy(tgt_hbm.at[pl.ds(e0, CB)], tidx)
            cpA = pltpu.async_copy(xu_hbm.at[sidx], abuf, sem1)
            cpB = pltpu.async_copy(xw_hbm.at[tidx], bbuf, sem2)
            pltpu.sync_copy(ep_hbm.at[pl.ds(e0, CB), :], ebuf)

            def bro(q, c):
                sv = sidx[pl.ds(q * _LANES, _LANES)]
                tv = tidx[pl.ds(q * _LANES, _LANES)]
                for j in range(_LANES):
                    r = q * _LANES + j
                    sbro[r, :] = jnp.full((_LANES,), sv[j], dtype=jnp.int32)
                    tbro[r, :] = jnp.full((_LANES,), tv[j], dtype=jnp.int32)
                return c

            lax.fori_loop(0, CB // _LANES, bro, 0)
            pltpu.sync_copy(sbro, srep_hbm.at[pl.ds(e0, CB), :])
            pltpu.sync_copy(tbro, trep_hbm.at[pl.ds(e0, CB), :])
            cpA.wait()
            cpB.wait()

            def row(r, c):
                for cc in range(d // _LANES):
                    sl = pl.ds(cc * _LANES, _LANES)
                    v = ebuf[r, sl] + abuf[r, sl] + bbuf[r, sl]
                    ebuf[r, sl] = jnp.maximum(v, 0.01 * v)
                return c

            lax.fori_loop(0, CB, row, 0)
            pltpu.sync_copy(ebuf, msg_hbm.at[pl.ds(e0, CB), :])
            return carry

        lax.fori_loop(0, nch, chunk, 0)

    return k(e_part, xu, xw, src, tgt)


def _logits_body(m_ref, wa_ref, out_ref, gmax_ref):
    i = pl.program_id(0)
    lb = lax.dot_general(m_ref[...], wa_ref[...], (((1,), (1,)), ((), ())),
                         preferred_element_type=jnp.float32)
    out_ref[...] = lb
    bm = jnp.full((8, 128), jnp.max(lb), dtype=jnp.float32)

    @pl.when(i == 0)
    def _():
        gmax_ref[...] = jnp.full((8, 128), -jnp.inf, dtype=jnp.float32)

    gmax_ref[...] = jnp.maximum(gmax_ref[...], bm)


def _stage_c(messages, W_attn):
    E, d = messages.shape
    hd = W_attn.shape[0]
    be = 2000
    return pl.pallas_call(
        _logits_body,
        grid=(E // be,),
        in_specs=[
            pl.BlockSpec((be, d), lambda i: (i, 0)),
            pl.BlockSpec((hd, d), lambda i: (0, 0)),
        ],
        out_specs=[
            pl.BlockSpec((be, hd), lambda i: (i, 0)),
            pl.BlockSpec((8, 128), lambda i: (0, 0)),
        ],
        out_shape=[
            jax.ShapeDtypeStruct((E, hd), jnp.float32),
            jax.ShapeDtypeStruct((8, 128), jnp.float32),
        ],
    )(messages, W_attn)


def _stage_d(lp3, srep, gv, n_nodes):
    ng, E = lp3.shape[0], lp3.shape[1]
    half = n_nodes // 2
    CD = 1000
    nch = E // CD

    @functools.partial(
        pl.kernel,
        out_type=jax.ShapeDtypeStruct((ng, n_nodes, _LANES), jnp.float32),
        mesh=_mesh(),
        compiler_params=_SC_PARAMS,
        scratch_types=[
            pltpu.VMEM((CD, _LANES), jnp.int32),
            pltpu.VMEM((CD, _LANES), jnp.float32),
            pltpu.VMEM((half, _LANES), jnp.float32),
            pltpu.VMEM((_LANES,), jnp.float32),
        ],
    )
    def k(lp_hbm, srep_hbm, gv_hbm, out_hbm, sbro, lbuf, acc, gvb):
        g = _wid()
        pltpu.sync_copy(gv_hbm, gvb)
        gvec = gvb[...]
        cols = lax.iota(jnp.int32, _LANES)
        half_u = jnp.full((_LANES,), half, dtype=jnp.uint32)
        for r in range(2):
            lo_v = jnp.full((_LANES,), r * half, dtype=jnp.int32)

            def zero(i, c):
                for u in range(8):
                    acc[i * 8 + u, :] = jnp.zeros((_LANES,), jnp.float32)
                return c

            lax.fori_loop(0, half // 8, zero, 0)

            def chunk(i, carry):
                e0 = i * CD
                pltpu.sync_copy(srep_hbm.at[pl.ds(e0, CD), :], sbro)
                pltpu.sync_copy(lp_hbm.at[g, pl.ds(e0, CD), :], lbuf)

                def edge(q, c):
                    for u in range(8):
                        row = q * 8 + u
                        rows = sbro[row, :] - lo_v
                        msk = lax.bitcast_convert_type(rows, jnp.uint32) < half_u
                        w = jnp.exp(lbuf[row, :] - gvec)
                        plsc.addupdate_scatter(acc, [rows, cols], w, mask=msk)
                    return c

                lax.fori_loop(0, CD // 8, edge, 0)
                return carry

            lax.fori_loop(0, nch, chunk, 0)

            def inv(i, c):
                acc[i, :] = 1.0 / jnp.maximum(acc[i, :], 1e-38)
                return c

            lax.fori_loop(0, half, inv, 0)
            pltpu.sync_copy(acc, out_hbm.at[g, pl.ds(r * half, half), :])

    return k(lp3, srep, gv)


def _stage_e(lp3, mp3, src, trep, ssum_flat, gv, n_nodes):
    ng, E = lp3.shape[0], lp3.shape[1]
    nmg = mp3.shape[0]
    half = n_nodes // 2
    CE = 512
    NSUB = CE // 128
    nch = E // CE

    @functools.partial(
        pl.kernel,
        out_type=jax.ShapeDtypeStruct((ng, n_nodes, _LANES), jnp.float32),
        mesh=_mesh(),
        compiler_params=_SC_PARAMS,
        scratch_types=[
            pltpu.VMEM((CE,), jnp.int32),
            pltpu.VMEM((CE, _LANES), jnp.int32),
            pltpu.VMEM((NSUB, 128), jnp.int32),
            pltpu.VMEM((CE, _LANES), jnp.float32),
            pltpu.VMEM((CE, _LANES), jnp.float32),
            pltpu.VMEM((CE, _LANES), jnp.float32),
            pltpu.VMEM((half, _LANES), jnp.float32),
            pltpu.VMEM((_LANES,), jnp.float32),
            pltpu.SemaphoreType.DMA,
        ],
    )
    def k(lp_hbm, mp_hbm, src_hbm, trep_hbm, ss_hbm, gv_hbm, out_hbm,
          sidx, tbro, gidx, lbuf, mbuf, sbuf, acc, gvb, sem):
        g = _wid()
        mg = lax.rem(g, nmg)
        pltpu.sync_copy(gv_hbm, gvb)
        gvec = gvb[...]
        cols = lax.iota(jnp.int32, _LANES)
        goff = jnp.full((_LANES,), g * n_nodes, dtype=jnp.int32)
        half_u = jnp.full((_LANES,), half, dtype=jnp.uint32)
        for r in range(2):
            lo_v = jnp.full((_LANES,), r * half, dtype=jnp.int32)

            def zero(i, c):
                for u in range(8):
                    acc[i * 8 + u, :] = jnp.zeros((_LANES,), jnp.float32)
                return c

            lax.fori_loop(0, half // 8, zero, 0)

            def chunk(i, carry):
                e0 = i * CE
                pltpu.sync_copy(src_hbm.at[pl.ds(e0, CE)], sidx)
                pltpu.sync_copy(trep_hbm.at[pl.ds(e0, CE), :], tbro)

                def mkidx(q, c):
                    iv = sidx[pl.ds(q * _LANES, _LANES)] + goff
                    qq = q // 8
                    gidx[qq, pl.ds((q - qq * 8) * _LANES, _LANES)] = iv
                    return c

                lax.fori_loop(0, CE // _LANES, mkidx, 0)
                cps = [pltpu.async_copy(ss_hbm.at[gidx.at[s]],
                                        sbuf.at[pl.ds(s * 128, 128), :], sem)
                       for s in range(NSUB)]
                pltpu.sync_copy(lp_hbm.at[g, pl.ds(e0, CE), :], lbuf)
                pltpu.sync_copy(mp_hbm.at[mg, pl.ds(e0, CE), :], mbuf)
                for cp in cps:
                    cp.wait()

                def edge(q, c):
                    for u in range(8):
                        row = q * 8 + u
                        rows = tbro[row, :] - lo_v
                        msk = lax.bitcast_convert_type(rows, jnp.uint32) < half_u
                        w = jnp.exp(lbuf[row, :] - gvec)
                        val = w * sbuf[row, :] * mbuf[row, :]
                        plsc.addupdate_scatter(acc, [rows, cols], val, mask=msk)
                    return c

                lax.fori_loop(0, CE // 8, edge, 0)
                return carry

            lax.fori_loop(0, nch, chunk, 0)
            pltpu.sync_copy(acc, out_hbm.at[g, pl.ds(r * half, half), :])

    return k(lp3, mp3, src, trep, ssum_flat, gv)


def _gru_body(agg_ref, x_ref, gx_ref, whh_ref, bhh_ref, h_ref):
    d = x_ref.shape[1]
    agg = agg_ref[...]
    a = 0.25 * (agg[:, :d] + agg[:, d:2 * d] + agg[:, 2 * d:3 * d]
                + agg[:, 3 * d:])
    gh = lax.dot_general(a, whh_ref[...], (((1,), (1,)), ((), ())),
                         preferred_element_type=jnp.float32) + bhh_ref[...]
    gx = gx_ref[...]
    rg = jax.nn.sigmoid(gx[:, :d] + gh[:, :d])
    z = jax.nn.sigmoid(gx[:, d:2 * d] + gh[:, d:2 * d])
    nn = jnp.tanh(gx[:, 2 * d:] + rg * gh[:, 2 * d:])
    h_ref[...] = (1.0 - z) * nn + z * a


def _stage_f(agg, x, gx, W_hh, b_hh):
    n, d = x.shape
    bn = 2000
    return pl.pallas_call(
        _gru_body,
        grid=(n // bn,),
        in_specs=[
            pl.BlockSpec((bn, 4 * d), lambda i: (i, 0)),
            pl.BlockSpec((bn, d), lambda i: (i, 0)),
            pl.BlockSpec((bn, 3 * d), lambda i: (i, 0)),
            pl.BlockSpec((3 * d, d), lambda i: (0, 0)),
            pl.BlockSpec((1, 3 * d), lambda i: (0, 0)),
        ],
        out_specs=pl.BlockSpec((bn, d), lambda i: (i, 0)),
        out_shape=jax.ShapeDtypeStruct((n, d), jnp.float32),
    )(agg, x, gx, W_hh, b_hh.reshape(1, 3 * d))


def kernel(x, edge_index, edge_attr, batch, W_e, W_hu, W_hw, W_attn,
           W_ih, W_hh, b_ih, b_hh):
    n, d = x.shape
    E = edge_attr.shape[0]
    hd = W_attn.shape[0]
    src = edge_index[0]
    tgt = edge_index[1]

    xu, xw, gx = _stage_a(x, W_hu, W_hw, W_ih, b_ih)
    e_part = _stage_a2(edge_attr, W_e)
    messages, srep, trep = _stage_b(e_part, xu, xw, src, tgt)
    logits, gmax = _stage_c(messages, W_attn)
    gv = jnp.full((_LANES,), jnp.max(gmax), dtype=jnp.float32)

    lt = logits.reshape(E, hd // _LANES, _LANES).transpose(1, 0, 2)
    mt = messages.reshape(E, d // _LANES, _LANES).transpose(1, 0, 2)
    rsum_l = _stage_d(lt, srep, gv, n)
    agg_l = _stage_e(lt, mt, src, trep, rsum_l.reshape(-1, _LANES), gv, n)
    agg = agg_l.transpose(1, 0, 2).reshape(n, hd)
    h = _stage_f(agg, x, gx, W_hh, b_hh)
    return (h, messages)


# speedup vs baseline: 1.2947x; 1.2947x over previous
import functools

import jax
import jax.numpy as jnp
from jax import lax
from jax.experimental import pallas as pl
from jax.experimental.pallas import tpu as pltpu, tpu_sc as plsc

_SC_PARAMS = pltpu.CompilerParams(use_tc_tiling_on_sc=False,
                                  needs_layout_passes=False)
_NW = 32
_LANES = 16


def _wid():
    return lax.axis_index("s") * 2 + lax.axis_index("c")


def _mesh():
    return plsc.VectorSubcoreMesh(core_axis_name="c", subcore_axis_name="s")


def _node_body(x_ref, whu_ref, whw_ref, wih_ref, bih_ref,
               xu_ref, xw_ref, gx_ref):
    x = x_ref[...]
    cdims = (((1,), (1,)), ((), ()))
    xu_ref[...] = lax.dot_general(x, whu_ref[...], cdims,
                                  preferred_element_type=jnp.float32)
    xw_ref[...] = lax.dot_general(x, whw_ref[...], cdims,
                                  preferred_element_type=jnp.float32)
    gx_ref[...] = lax.dot_general(x, wih_ref[...], cdims,
                                  preferred_element_type=jnp.float32) + bih_ref[...]


def _stage_a(x, W_hu, W_hw, W_ih, b_ih):
    n, d = x.shape
    bn = 2000
    grid = (n // bn,)
    return pl.pallas_call(
        _node_body,
        grid=grid,
        in_specs=[
            pl.BlockSpec((bn, d), lambda i: (i, 0)),
            pl.BlockSpec((d, d), lambda i: (0, 0)),
            pl.BlockSpec((d, d), lambda i: (0, 0)),
            pl.BlockSpec((3 * d, d), lambda i: (0, 0)),
            pl.BlockSpec((1, 3 * d), lambda i: (0, 0)),
        ],
        out_specs=[
            pl.BlockSpec((bn, d), lambda i: (i, 0)),
            pl.BlockSpec((bn, d), lambda i: (i, 0)),
            pl.BlockSpec((bn, 3 * d), lambda i: (i, 0)),
        ],
        out_shape=[
            jax.ShapeDtypeStruct((n, d), jnp.float32),
            jax.ShapeDtypeStruct((n, d), jnp.float32),
            jax.ShapeDtypeStruct((n, 3 * d), jnp.float32),
        ],
    )(x, W_hu, W_hw, W_ih, b_ih.reshape(1, 3 * d))


def _epart_body(ea_ref, we_ref, out_ref):
    out_ref[...] = lax.dot_general(ea_ref[...], we_ref[...],
                                   (((1,), (1,)), ((), ())),
                                   preferred_element_type=jnp.float32)


def _stage_a2(edge_attr, W_e):
    e, de = edge_attr.shape
    d = W_e.shape[0]
    be = 10000
    return pl.pallas_call(
        _epart_body,
        grid=(e // be,),
        in_specs=[
            pl.BlockSpec((be, de), lambda i: (i, 0)),
            pl.BlockSpec((d, de), lambda i: (0, 0)),
        ],
        out_specs=pl.BlockSpec((be, d), lambda i: (i, 0)),
        out_shape=jax.ShapeDtypeStruct((e, d), jnp.float32),
    )(edge_attr, W_e)


def _stage_b(e_part, xu, xw, src, tgt):
    E, d = e_part.shape
    epw = E // _NW
    CB = 80
    nch = epw // CB

    @functools.partial(
        pl.kernel,
        out_type=[
            jax.ShapeDtypeStruct((E, d), jnp.float32),
            jax.ShapeDtypeStruct((E, _LANES), jnp.int32),
            jax.ShapeDtypeStruct((E, _LANES), jnp.int32),
        ],
        mesh=_mesh(),
        compiler_params=_SC_PARAMS,
        scratch_types=[
            pltpu.VMEM((CB,), jnp.int32),
            pltpu.VMEM((CB,), jnp.int32),
            pltpu.VMEM((CB, d), jnp.float32),
            pltpu.VMEM((CB, d), jnp.float32),
            pltpu.VMEM((CB, d), jnp.float32),
            pltpu.VMEM((CB, _LANES), jnp.int32),
            pltpu.VMEM((CB, _LANES), jnp.int32),
            pltpu.SemaphoreType.DMA,
            pltpu.SemaphoreType.DMA,
        ],
    )
    def k(ep_hbm, xu_hbm, xw_hbm, src_hbm, tgt_hbm,
          msg_hbm, srep_hbm, trep_hbm,
          sidx, tidx, ebuf, abuf, bbuf, sbro, tbro, sem1, sem2):
        base = _wid() * epw

        def chunk(i, carry):
            e0 = base + i * CB
            pltpu.sync_copy(src_hbm.at[pl.ds(e0, CB)], sidx)
            pltpu.sync_copy(tgt_hbm.at[pl.ds(e0, CB)], tidx)
            cpA = pltpu.async_copy(xu_hbm.at[sidx], abuf, sem1)
            cpB = pltpu.async_copy(xw_hbm.at[tidx], bbuf, sem2)
            pltpu.sync_copy(ep_hbm.at[pl.ds(e0, CB), :], ebuf)

            def bro(q, c):
                sv = sidx[pl.ds(q * _LANES, _LANES)]
                tv = tidx[pl.ds(q * _LANES, _LANES)]
                for j in range(_LANES):
                    r = q * _LANES + j
                    sbro[r, :] = jnp.full((_LANES,), sv[j], dtype=jnp.int32)
                    tbro[r, :] = jnp.full((_LANES,), tv[j], dtype=jnp.int32)
                return c

            lax.fori_loop(0, CB // _LANES, bro, 0)
            pltpu.sync_copy(sbro, srep_hbm.at[pl.ds(e0, CB), :])
            pltpu.sync_copy(tbro, trep_hbm.at[pl.ds(e0, CB), :])
            cpA.wait()
            cpB.wait()

            def row(r, c):
                for cc in range(d // _LANES):
                    sl = pl.ds(cc * _LANES, _LANES)
                    v = ebuf[r, sl] + abuf[r, sl] + bbuf[r, sl]
                    ebuf[r, sl] = jnp.maximum(v, 0.01 * v)
                return c

            lax.fori_loop(0, CB, row, 0)
            pltpu.sync_copy(ebuf, msg_hbm.at[pl.ds(e0, CB), :])
            return carry

        lax.fori_loop(0, nch, chunk, 0)

    return k(e_part, xu, xw, src, tgt)


def _logits_body(m_ref, wa_ref, out_ref, gmax_ref):
    i = pl.program_id(0)
    lb = lax.dot_general(m_ref[...], wa_ref[...], (((1,), (1,)), ((), ())),
                         preferred_element_type=jnp.float32)
    out_ref[...] = lb
    bm = jnp.full((8, 128), jnp.max(lb), dtype=jnp.float32)

    @pl.when(i == 0)
    def _():
        gmax_ref[...] = jnp.full((8, 128), -jnp.inf, dtype=jnp.float32)

    gmax_ref[...] = jnp.maximum(gmax_ref[...], bm)


def _stage_c(messages, W_attn):
    E, d = messages.shape
    hd = W_attn.shape[0]
    be = 2000
    return pl.pallas_call(
        _logits_body,
        grid=(E // be,),
        in_specs=[
            pl.BlockSpec((be, d), lambda i: (i, 0)),
            pl.BlockSpec((hd, d), lambda i: (0, 0)),
        ],
        out_specs=[
            pl.BlockSpec((be, hd), lambda i: (i, 0)),
            pl.BlockSpec((8, 128), lambda i: (0, 0)),
        ],
        out_shape=[
            jax.ShapeDtypeStruct((E, hd), jnp.float32),
            jax.ShapeDtypeStruct((8, 128), jnp.float32),
        ],
    )(messages, W_attn)


def _expw_body(l_ref, gm_ref, w_ref):
    w_ref[...] = jnp.exp(l_ref[...] - gm_ref[0, 0])


def _stage_w(logits, gmax):
    E, hd = logits.shape
    be = 2000
    return pl.pallas_call(
        _expw_body,
        grid=(E // be,),
        in_specs=[
            pl.BlockSpec((be, hd), lambda i: (i, 0)),
            pl.BlockSpec((8, 128), lambda i: (0, 0)),
        ],
        out_specs=pl.BlockSpec((be, hd), lambda i: (i, 0)),
        out_shape=jax.ShapeDtypeStruct((E, hd), jnp.float32),
    )(logits, gmax)


def _stage_d(lp3, srep, gv, n_nodes):
    ng, E = lp3.shape[0], lp3.shape[1]
    half = n_nodes // 2
    CD = 1000
    nch = E // CD

    @functools.partial(
        pl.kernel,
        out_type=jax.ShapeDtypeStruct((ng, n_nodes, _LANES), jnp.float32),
        mesh=_mesh(),
        compiler_params=_SC_PARAMS,
        scratch_types=[
            pltpu.VMEM((CD, _LANES), jnp.int32),
            pltpu.VMEM((CD, _LANES), jnp.float32),
            pltpu.VMEM((half, _LANES), jnp.float32),
            pltpu.VMEM((_LANES,), jnp.float32),
        ],
    )
    def k(lp_hbm, srep_hbm, gv_hbm, out_hbm, sbro, lbuf, acc, gvb):
        g = _wid()
        pltpu.sync_copy(gv_hbm, gvb)
        gvec = gvb[...]
        cols = lax.iota(jnp.int32, _LANES)
        half_u = jnp.full((_LANES,), half, dtype=jnp.uint32)
        for r in range(2):
            lo_v = jnp.full((_LANES,), r * half, dtype=jnp.int32)

            def zero(i, c):
                for u in range(8):
                    acc[i * 8 + u, :] = jnp.zeros((_LANES,), jnp.float32)
                return c

            lax.fori_loop(0, half // 8, zero, 0)

            def chunk(i, carry):
                e0 = i * CD
                pltpu.sync_copy(srep_hbm.at[pl.ds(e0, CD), :], sbro)
                pltpu.sync_copy(lp_hbm.at[g, pl.ds(e0, CD), :], lbuf)

                def edge(q, c):
                    for u in range(8):
                        row = q * 8 + u
                        rows = sbro[row, :] - lo_v
                        msk = lax.bitcast_convert_type(rows, jnp.uint32) < half_u
                        w = lbuf[row, :]
                        plsc.addupdate_scatter(acc, [rows, cols], w, mask=msk)
                    return c

                lax.fori_loop(0, CD // 8, edge, 0)
                return carry

            lax.fori_loop(0, nch, chunk, 0)

            def inv(i, c):
                acc[i, :] = 1.0 / jnp.maximum(acc[i, :], 1e-38)
                return c

            lax.fori_loop(0, half, inv, 0)
            pltpu.sync_copy(acc, out_hbm.at[g, pl.ds(r * half, half), :])

    return k(lp3, srep, gv)


def _stage_e(lp3, mp3, src, trep, ssum_flat, gv, n_nodes):
    ng, E = lp3.shape[0], lp3.shape[1]
    nmg = mp3.shape[0]
    half = n_nodes // 2
    CE = 512
    NSUB = CE // 128
    nch = E // CE

    @functools.partial(
        pl.kernel,
        out_type=jax.ShapeDtypeStruct((ng, n_nodes, _LANES), jnp.float32),
        mesh=_mesh(),
        compiler_params=_SC_PARAMS,
        scratch_types=[
            pltpu.VMEM((CE,), jnp.int32),
            pltpu.VMEM((CE, _LANES), jnp.int32),
            pltpu.VMEM((NSUB, 128), jnp.int32),
            pltpu.VMEM((CE, _LANES), jnp.float32),
            pltpu.VMEM((CE, _LANES), jnp.float32),
            pltpu.VMEM((CE, _LANES), jnp.float32),
            pltpu.VMEM((half, _LANES), jnp.float32),
            pltpu.VMEM((_LANES,), jnp.float32),
            pltpu.SemaphoreType.DMA,
        ],
    )
    def k(lp_hbm, mp_hbm, src_hbm, trep_hbm, ss_hbm, gv_hbm, out_hbm,
          sidx, tbro, gidx, lbuf, mbuf, sbuf, acc, gvb, sem):
        g = _wid()
        mg = lax.rem(g, nmg)
        pltpu.sync_copy(gv_hbm, gvb)
        gvec = gvb[...]
        cols = lax.iota(jnp.int32, _LANES)
        goff = jnp.full((_LANES,), g * n_nodes, dtype=jnp.int32)
        half_u = jnp.full((_LANES,), half, dtype=jnp.uint32)
        for r in range(2):
            lo_v = jnp.full((_LANES,), r * half, dtype=jnp.int32)

            def zero(i, c):
                for u in range(8):
                    acc[i * 8 + u, :] = jnp.zeros((_LANES,), jnp.float32)
                return c

            lax.fori_loop(0, half // 8, zero, 0)

            def chunk(i, carry):
                e0 = i * CE
                pltpu.sync_copy(src_hbm.at[pl.ds(e0, CE)], sidx)
                pltpu.sync_copy(trep_hbm.at[pl.ds(e0, CE), :], tbro)

                def mkidx(q, c):
                    iv = sidx[pl.ds(q * _LANES, _LANES)] + goff
                    qq = q // 8
                    gidx[qq, pl.ds((q - qq * 8) * _LANES, _LANES)] = iv
                    return c

                lax.fori_loop(0, CE // _LANES, mkidx, 0)
                cps = [pltpu.async_copy(ss_hbm.at[gidx.at[s]],
                                        sbuf.at[pl.ds(s * 128, 128), :], sem)
                       for s in range(NSUB)]
                pltpu.sync_copy(lp_hbm.at[g, pl.ds(e0, CE), :], lbuf)
                pltpu.sync_copy(mp_hbm.at[mg, pl.ds(e0, CE), :], mbuf)
                for cp in cps:
                    cp.wait()

                def edge(q, c):
                    for u in range(8):
                        row = q * 8 + u
                        rows = tbro[row, :] - lo_v
                        msk = lax.bitcast_convert_type(rows, jnp.uint32) < half_u
                        val = lbuf[row, :] * sbuf[row, :] * mbuf[row, :]
                        plsc.addupdate_scatter(acc, [rows, cols], val, mask=msk)
                    return c

                lax.fori_loop(0, CE // 8, edge, 0)
                return carry

            lax.fori_loop(0, nch, chunk, 0)
            pltpu.sync_copy(acc, out_hbm.at[g, pl.ds(r * half, half), :])

    return k(lp3, mp3, src, trep, ssum_flat, gv)


def _gru_body(agg_ref, x_ref, gx_ref, whh_ref, bhh_ref, h_ref):
    d = x_ref.shape[1]
    agg = agg_ref[...]
    a = 0.25 * (agg[:, :d] + agg[:, d:2 * d] + agg[:, 2 * d:3 * d]
                + agg[:, 3 * d:])
    gh = lax.dot_general(a, whh_ref[...], (((1,), (1,)), ((), ())),
                         preferred_element_type=jnp.float32) + bhh_ref[...]
    gx = gx_ref[...]
    rg = jax.nn.sigmoid(gx[:, :d] + gh[:, :d])
    z = jax.nn.sigmoid(gx[:, d:2 * d] + gh[:, d:2 * d])
    nn = jnp.tanh(gx[:, 2 * d:] + rg * gh[:, 2 * d:])
    h_ref[...] = (1.0 - z) * nn + z * a


def _stage_f(agg, x, gx, W_hh, b_hh):
    n, d = x.shape
    bn = 2000
    return pl.pallas_call(
        _gru_body,
        grid=(n // bn,),
        in_specs=[
            pl.BlockSpec((bn, 4 * d), lambda i: (i, 0)),
            pl.BlockSpec((bn, d), lambda i: (i, 0)),
            pl.BlockSpec((bn, 3 * d), lambda i: (i, 0)),
            pl.BlockSpec((3 * d, d), lambda i: (0, 0)),
            pl.BlockSpec((1, 3 * d), lambda i: (0, 0)),
        ],
        out_specs=pl.BlockSpec((bn, d), lambda i: (i, 0)),
        out_shape=jax.ShapeDtypeStruct((n, d), jnp.float32),
    )(agg, x, gx, W_hh, b_hh.reshape(1, 3 * d))


def kernel(x, edge_index, edge_attr, batch, W_e, W_hu, W_hw, W_attn,
           W_ih, W_hh, b_ih, b_hh):
    n, d = x.shape
    E = edge_attr.shape[0]
    hd = W_attn.shape[0]
    src = edge_index[0]
    tgt = edge_index[1]

    xu, xw, gx = _stage_a(x, W_hu, W_hw, W_ih, b_ih)
    e_part = _stage_a2(edge_attr, W_e)
    messages, srep, trep = _stage_b(e_part, xu, xw, src, tgt)
    logits, gmax = _stage_c(messages, W_attn)
    gv = jnp.full((_LANES,), jnp.max(gmax), dtype=jnp.float32)

    wexp = _stage_w(logits, gmax)
    lt = wexp.reshape(E, hd // _LANES, _LANES).transpose(1, 0, 2)
    mt = messages.reshape(E, d // _LANES, _LANES).transpose(1, 0, 2)
    rsum_l = _stage_d(lt, srep, gv, n)
    agg_l = _stage_e(lt, mt, src, trep, rsum_l.reshape(-1, _LANES), gv, n)
    agg = agg_l.transpose(1, 0, 2).reshape(n, hd)
    h = _stage_f(agg, x, gx, W_hh, b_hh)
    return (h, messages)
